# baseline (speedup 1.0000x reference)
import functools
import math

import jax
import jax.numpy as jnp
from jax import lax
from jax.experimental import pallas as pl
from jax.experimental.pallas import tpu as pltpu
from jax.experimental.pallas import tpu_sc as plsc

_TOP_K = 5
_W1, _W2, _W3 = 0.5, 0.3, 0.2
_EPS = 1e-12
_NEG = -1e30
_IMAX = 2**31 - 1

_KB = 2048
_CB = 1024
_GR = 16
_RPC = _CB // _GR
_GW = _TOP_K * _RPC


def _normalize(x):
    n = jnp.sqrt(jnp.sum(x * x, axis=-1, keepdims=True))
    return x / jnp.maximum(n, 1e-12)


def _top1(vals, ids):
    v = jnp.max(vals, axis=-1, keepdims=True)
    cand = jnp.where(vals >= v, ids, _IMAX)
    ix = jnp.min(cand, axis=-1, keepdims=True)
    return v, ix


def _pass1_body(q_ref, k_ref, m_o, s_o, r2_o, cand_o,
                m_s, l_s, r2_s, bm_s, *, kb, nkb, kk):
    i = pl.program_id(0)
    n = q_ref.shape[0]

    @pl.when(i == 0)
    def _init():
        m_s[...] = jnp.full_like(m_s[...], _NEG)
        l_s[...] = jnp.zeros_like(l_s[...])
        r2_s[...] = jnp.full_like(r2_s[...], _NEG)
        bm_s[...] = jnp.full_like(bm_s[...], _NEG)

    qn = _normalize(q_ref[...])
    k = k_ref[...]
    rows_left = kk - i * kb
    rmask = lax.broadcasted_iota(jnp.int32, (kb, 1), 0) < rows_left
    k = jnp.where(rmask, k, 0.0)
    kn = _normalize(k)
    s = lax.dot_general(qn, kn, (((1,), (1,)), ((), ())),
                        preferred_element_type=jnp.float32)
    col = lax.broadcasted_iota(jnp.int32, (n, kb), 1) + i * kb
    s = jnp.where(col < kk, s, _NEG)

    bm0 = jnp.max(s[:, :_CB], axis=-1, keepdims=True)
    bm1 = jnp.max(s[:, _CB:], axis=-1, keepdims=True)
    lane = lax.broadcasted_iota(jnp.int32, (n, 128), 1)
    bm = bm_s[...]
    bm = jnp.where(lane == 2 * i, jnp.broadcast_to(bm0, bm.shape), bm)
    bm = jnp.where(lane == 2 * i + 1, jnp.broadcast_to(bm1, bm.shape), bm)
    bm_s[...] = bm

    m_old = m_s[...][:, :1]
    bmax = jnp.maximum(bm0, bm1)
    m_new = jnp.maximum(m_old, bmax)
    alpha = jnp.exp(m_old - m_new)
    p = jnp.exp(s - m_new)
    l_new = l_s[...][:, :1] * alpha + jnp.sum(p, axis=-1, keepdims=True)

    m_s[...] = jnp.broadcast_to(m_new, m_s.shape)
    l_s[...] = jnp.broadcast_to(l_new, l_s.shape)

    eq = s >= bmax
    v2m = jnp.max(jnp.where(eq, _NEG, s), axis=-1, keepdims=True)
    cnt = jnp.sum(jnp.where(eq, 1.0, 0.0), axis=-1, keepdims=True)
    v2b = jnp.where(cnt > 1.0, bmax, v2m)
    r2_old = r2_s[...][:, :1]
    r2_new = jnp.maximum(jnp.minimum(m_old, bmax), jnp.maximum(r2_old, v2b))
    r2_s[...] = jnp.broadcast_to(r2_new, r2_s.shape)

    @pl.when(i == nkb - 1)
    def _fin():
        m_o[...] = jnp.broadcast_to(m_new, m_o.shape)
        s_o[...] = jnp.broadcast_to(l_new, s_o.shape)
        r2_o[...] = jnp.broadcast_to(r2_new, r2_o.shape)

        bvals = bm_s[...]
        bids = lane
        sel = []
        for _ in range(_TOP_K):
            _, ix = _top1(bvals, bids)
            sel.append(ix)
            bvals = jnp.where(bids == ix, _NEG, bvals)
        cand = jnp.zeros((n, 128), jnp.int32)
        for j in range(_TOP_K):
            cand = jnp.where(lane == j, jnp.broadcast_to(sel[j], (n, 128)),
                             cand)
        cand_o[...] = cand


def _pass2_body(q_ref, k_ref, m_ref, s_ref, r2_ref, attn_o, msem_o, conf_o,
                t_s, acc_s, *, kb, nkb, kk):
    i = pl.program_id(0)
    n = q_ref.shape[0]

    @pl.when(i == 0)
    def _init():
        t_s[...] = jnp.zeros_like(t_s[...])
        acc_s[...] = jnp.zeros_like(acc_s[...])

    qn = _normalize(q_ref[...])
    k = k_ref[...]
    rows_left = kk - i * kb
    rmask = lax.broadcasted_iota(jnp.int32, (kb, 1), 0) < rows_left
    k = jnp.where(rmask, k, 0.0)
    kn = _normalize(k)
    s = lax.dot_general(qn, kn, (((1,), (1,)), ((), ())),
                        preferred_element_type=jnp.float32)
    col = lax.broadcasted_iota(jnp.int32, (n, kb), 1) + i * kb
    s = jnp.where(col < kk, s, _NEG)
    m = m_ref[...][:, :1]
    big_s = jnp.maximum(s_ref[...][:, :1], 1e-30)
    recip = 1.0 / big_s
    e = jnp.exp(s - m)
    attn_o[...] = e * recip
    t_new = t_s[...][:, :1] + jnp.sum(e * s, axis=-1, keepdims=True)
    acc_new = acc_s[...] + lax.dot_general(
        e, kn, (((1,), (0,)), ((), ())), preferred_element_type=jnp.float32)
    t_s[...] = jnp.broadcast_to(t_new, t_s.shape)
    acc_s[...] = acc_new

    @pl.when(i == nkb - 1)
    def _fin():
        msem_o[...] = acc_new * recip
        max_attn = recip
        entropy = m + jnp.log(big_s) - t_new * recip
        max_entropy = math.log(kk + _EPS)
        norm_ent = jnp.clip(entropy / (max_entropy + _EPS), 0.0, 1.0)
        gap = (1.0 - jnp.exp(r2_ref[...][:, :1] - m)) * recip
        conf = jnp.clip(_W1 * max_attn + _W2 * (1.0 - norm_ent) + _W3 * gap,
                        0.0, 1.0)
        conf_o[...] = jnp.broadcast_to(conf, conf_o.shape)


def _pass3_body(cv_ref, cand_ref, tail_ref, topk_o, *, kk):
    n = cv_ref.shape[0]
    w = cv_ref.shape[1]
    c = lax.broadcasted_iota(jnp.int32, (n, w), 1)
    cb = c // _CB
    cw = c - cb * _CB
    cand_sel = jnp.zeros((n, w), jnp.int32)
    for j in range(_TOP_K):
        cand_sel = jnp.where(
            cb == j, jnp.broadcast_to(cand_ref[...][:, j:j + 1], (n, w)),
            cand_sel)
    clamp = ((kk - _CB) // 128) * 128
    cand_sel = jnp.clip(cand_sel, 0, kk // _CB)
    gid = jnp.minimum(cand_sel * _CB, clamp) + cw
    vals = cv_ref[...]
    lane = lax.broadcasted_iota(jnp.int32, (n, 128), 1)
    tcol = (kk - 1) // 128
    tgid = tcol * 128 + lane
    tvals = jnp.where(tgid < kk, tail_ref[...], _NEG)
    vals = jnp.concatenate([vals, tvals], axis=1)
    gid = jnp.concatenate([gid, tgid], axis=1)
    out = jnp.zeros((n, 128), jnp.int32)
    for j in range(_TOP_K):
        _, ix = _top1(vals, gid)
        out = jnp.where(lane == j, jnp.broadcast_to(ix, (n, 128)), out)
        vals = jnp.where(gid == ix, _NEG, vals)
    topk_o[...] = out


def _sc_gather(cand, attn, n, kk):
    info = plsc.get_sparse_core_info()
    nw = info.num_cores * info.num_subcores
    rows_per_w = n // nw
    mesh = plsc.VectorSubcoreMesh(core_axis_name="c", subcore_axis_name="s")
    cand3 = cand.reshape(n, 1, 128)

    clamp = ((kk - _CB) // 128) * 128

    @functools.partial(
        pl.kernel, mesh=mesh,
        out_type=jax.ShapeDtypeStruct((n, _TOP_K, _CB), jnp.float32),
        scratch_types=[
            pltpu.VMEM((rows_per_w, 1, 128), jnp.int32),
            pltpu.VMEM((_TOP_K, 8, _CB), jnp.float32),
            pltpu.VMEM((_TOP_K, 8, _CB), jnp.float32),
            pltpu.VMEM((_TOP_K, _CB), jnp.float32),
            pltpu.SemaphoreType.DMA,
            pltpu.SemaphoreType.DMA,
        ],
    )
    def k(cand_hbm, attn_hbm, out_hbm, cand_v, vals_a, vals_b, row_v,
          sem_a, sem_b):
        wid = lax.axis_index("s") * info.num_cores + lax.axis_index("c")
        base = wid * rows_per_w
        pltpu.sync_copy(cand_hbm.at[pl.ds(base, rows_per_w)], cand_v)

        def start_row(t, buf, sem):
            r = base + t
            g8 = pl.multiple_of((r // 8) * 8, 8)
            cvec = jnp.maximum(cand_v[t, 0, pl.ds(0, 16)], 0)
            for j in range(_TOP_K):
                start = pl.multiple_of(
                    jnp.minimum(jnp.minimum(cvec[j], kk // _CB) * _CB, clamp),
                    128)
                pltpu.make_async_copy(
                    attn_hbm.at[pl.ds(g8, 8), pl.ds(start, _CB)],
                    buf.at[j], sem).start()

        def wait_row(buf, sem):
            for j in range(_TOP_K):
                pltpu.make_async_copy(
                    attn_hbm.at[pl.ds(0, 8), pl.ds(0, _CB)],
                    buf.at[j], sem).wait()

        def finish_row(t, buf):
            r = base + t
            rm8 = r - (r // 8) * 8

            def extract(i, _):
                for j in range(_TOP_K):
                    row_v[j, pl.ds(i * 16, 16)] = buf[j, rm8,
                                                      pl.ds(i * 16, 16)]
                return _

            lax.fori_loop(0, _CB // 16, extract, 0)
            pltpu.sync_copy(row_v, out_hbm.at[r])

        start_row(0, vals_a, sem_a)

        def body(h, carry):
            t0 = 2 * h
            start_row(t0 + 1, vals_b, sem_b)
            wait_row(vals_a, sem_a)
            finish_row(t0, vals_a)

            @pl.when(t0 + 2 < rows_per_w)
            def _refill():
                start_row(t0 + 2, vals_a, sem_a)

            wait_row(vals_b, sem_b)
            finish_row(t0 + 1, vals_b)
            return carry

        lax.fori_loop(0, rows_per_w // 2, body, 0)

    return k(cand3, attn)


def kernel(query, keys):
    n, d = query.shape
    kk = keys.shape[0]
    nkb = pl.cdiv(kk, _KB)

    p1 = pl.pallas_call(
        functools.partial(_pass1_body, kb=_KB, nkb=nkb, kk=kk),
        grid=(nkb,),
        in_specs=[
            pl.BlockSpec((n, d), lambda i: (0, 0)),
            pl.BlockSpec((_KB, d), lambda i: (i, 0)),
        ],
        out_specs=[
            pl.BlockSpec((n, 128), lambda i: (0, 0)),
            pl.BlockSpec((n, 128), lambda i: (0, 0)),
            pl.BlockSpec((n, 128), lambda i: (0, 0)),
            pl.BlockSpec((n, 128), lambda i: (0, 0)),
        ],
        out_shape=[
            jax.ShapeDtypeStruct((n, 128), jnp.float32),
            jax.ShapeDtypeStruct((n, 128), jnp.float32),
            jax.ShapeDtypeStruct((n, 128), jnp.float32),
            jax.ShapeDtypeStruct((n, 128), jnp.int32),
        ],
        scratch_shapes=[
            pltpu.VMEM((n, 128), jnp.float32),
            pltpu.VMEM((n, 128), jnp.float32),
            pltpu.VMEM((n, 128), jnp.float32),
            pltpu.VMEM((n, 128), jnp.float32),
        ],
    )
    m_row, s_row, r2_row, cand_r = p1(query, keys)

    attn, m_sem, conf_r = pl.pallas_call(
        functools.partial(_pass2_body, kb=_KB, nkb=nkb, kk=kk),
        grid=(nkb,),
        in_specs=[
            pl.BlockSpec((n, d), lambda i: (0, 0)),
            pl.BlockSpec((_KB, d), lambda i: (i, 0)),
            pl.BlockSpec((n, 128), lambda i: (0, 0)),
            pl.BlockSpec((n, 128), lambda i: (0, 0)),
            pl.BlockSpec((n, 128), lambda i: (0, 0)),
        ],
        out_specs=[
            pl.BlockSpec((n, _KB), lambda i: (0, i)),
            pl.BlockSpec((n, d), lambda i: (0, 0)),
            pl.BlockSpec((n, 128), lambda i: (0, 0)),
        ],
        out_shape=[
            jax.ShapeDtypeStruct((n, kk), jnp.float32),
            jax.ShapeDtypeStruct((n, d), jnp.float32),
            jax.ShapeDtypeStruct((n, 128), jnp.float32),
        ],
        scratch_shapes=[
            pltpu.VMEM((n, 128), jnp.float32),
            pltpu.VMEM((n, d), jnp.float32),
        ],
    )(query, keys, m_row, s_row, r2_row)

    cand_vals = _sc_gather(cand_r, attn, n, kk).reshape(n, _TOP_K * _CB)

    rb = 128
    tcol = (kk - 1) // 128
    topk_r = pl.pallas_call(
        functools.partial(_pass3_body, kk=kk),
        grid=(n // rb,),
        in_specs=[
            pl.BlockSpec((rb, _TOP_K * _CB), lambda i: (i, 0)),
            pl.BlockSpec((rb, 128), lambda i: (i, 0)),
            pl.BlockSpec((rb, 128), lambda i: (i, tcol)),
        ],
        out_specs=pl.BlockSpec((rb, 128), lambda i: (i, 0)),
        out_shape=jax.ShapeDtypeStruct((n, 128), jnp.int32),
    )(cand_vals, cand_r, attn)

    conf = conf_r[:, 0]
    topk_idx = topk_r[:, :_TOP_K]
    return (m_sem, attn, conf, topk_idx)


# speedup vs baseline: 1.0171x; 1.0171x over previous
import functools
import math

import jax
import jax.numpy as jnp
from jax import lax
from jax.experimental import pallas as pl
from jax.experimental.pallas import tpu as pltpu
from jax.experimental.pallas import tpu_sc as plsc

_TOP_K = 5
_W1, _W2, _W3 = 0.5, 0.3, 0.2
_EPS = 1e-12
_NEG = -1e30
_IMAX = 2**31 - 1

_KB = 2048
_KB1 = 3072
_CB = 1024
_GR = 16
_RPC = _CB // _GR
_GW = _TOP_K * _RPC


def _normalize(x):
    n = jnp.sqrt(jnp.sum(x * x, axis=-1, keepdims=True))
    return x / jnp.maximum(n, 1e-12)


def _top1(vals, ids):
    v = jnp.max(vals, axis=-1, keepdims=True)
    cand = jnp.where(vals >= v, ids, _IMAX)
    ix = jnp.min(cand, axis=-1, keepdims=True)
    return v, ix


def _pass1_body(q_ref, k_ref, m_o, s_o, r2_o, cand_o,
                m_s, l_s, r2_s, bm_s, *, kb, nkb, kk):
    i = pl.program_id(0)
    n = q_ref.shape[0]

    @pl.when(i == 0)
    def _init():
        m_s[...] = jnp.full_like(m_s[...], _NEG)
        l_s[...] = jnp.zeros_like(l_s[...])
        r2_s[...] = jnp.full_like(r2_s[...], _NEG)
        bm_s[...] = jnp.full_like(bm_s[...], _NEG)

    qn = _normalize(q_ref[...])
    k = k_ref[...]
    rows_left = kk - i * kb
    rmask = lax.broadcasted_iota(jnp.int32, (kb, 1), 0) < rows_left
    k = jnp.where(rmask, k, 0.0)
    kn = _normalize(k)
    s = lax.dot_general(qn, kn, (((1,), (1,)), ((), ())),
                        preferred_element_type=jnp.float32)
    col = lax.broadcasted_iota(jnp.int32, (n, kb), 1) + i * kb
    s = jnp.where(col < kk, s, _NEG)

    nsub = kb // _CB
    bms = [jnp.max(s[:, j * _CB:(j + 1) * _CB], axis=-1, keepdims=True)
           for j in range(nsub)]
    lane = lax.broadcasted_iota(jnp.int32, (n, 128), 1)
    bm = bm_s[...]
    for j in range(nsub):
        bm = jnp.where(lane == nsub * i + j,
                       jnp.broadcast_to(bms[j], bm.shape), bm)
    bm_s[...] = bm

    m_old = m_s[...][:, :1]
    bmax = bms[0]
    for j in range(1, nsub):
        bmax = jnp.maximum(bmax, bms[j])
    m_new = jnp.maximum(m_old, bmax)
    alpha = jnp.exp(m_old - m_new)
    p = jnp.exp(s - m_new)
    l_new = l_s[...][:, :1] * alpha + jnp.sum(p, axis=-1, keepdims=True)

    m_s[...] = jnp.broadcast_to(m_new, m_s.shape)
    l_s[...] = jnp.broadcast_to(l_new, l_s.shape)

    eq = s >= bmax
    v2m = jnp.max(jnp.where(eq, _NEG, s), axis=-1, keepdims=True)
    cnt = jnp.sum(jnp.where(eq, 1.0, 0.0), axis=-1, keepdims=True)
    v2b = jnp.where(cnt > 1.0, bmax, v2m)
    r2_old = r2_s[...][:, :1]
    r2_new = jnp.maximum(jnp.minimum(m_old, bmax), jnp.maximum(r2_old, v2b))
    r2_s[...] = jnp.broadcast_to(r2_new, r2_s.shape)

    @pl.when(i == nkb - 1)
    def _fin():
        m_o[...] = jnp.broadcast_to(m_new, m_o.shape)
        s_o[...] = jnp.broadcast_to(l_new, s_o.shape)
        r2_o[...] = jnp.broadcast_to(r2_new, r2_o.shape)

        bvals = bm_s[...]
        bids = lane
        sel = []
        for _ in range(_TOP_K):
            _, ix = _top1(bvals, bids)
            sel.append(ix)
            bvals = jnp.where(bids == ix, _NEG, bvals)
        cand = jnp.zeros((n, 128), jnp.int32)
        for j in range(_TOP_K):
            cand = jnp.where(lane == j, jnp.broadcast_to(sel[j], (n, 128)),
                             cand)
        cand_o[...] = cand


def _pass2_body(q_ref, k_ref, m_ref, s_ref, r2_ref, attn_o, msem_o, conf_o,
                t_s, acc_s, *, kb, nkb, kk):
    i = pl.program_id(0)
    n = q_ref.shape[0]

    @pl.when(i == 0)
    def _init():
        t_s[...] = jnp.zeros_like(t_s[...])
        acc_s[...] = jnp.zeros_like(acc_s[...])

    qn = _normalize(q_ref[...])
    k = k_ref[...]
    rows_left = kk - i * kb
    rmask = lax.broadcasted_iota(jnp.int32, (kb, 1), 0) < rows_left
    k = jnp.where(rmask, k, 0.0)
    kn = _normalize(k)
    s = lax.dot_general(qn, kn, (((1,), (1,)), ((), ())),
                        preferred_element_type=jnp.float32)
    col = lax.broadcasted_iota(jnp.int32, (n, kb), 1) + i * kb
    s = jnp.where(col < kk, s, _NEG)
    m = m_ref[...][:, :1]
    big_s = jnp.maximum(s_ref[...][:, :1], 1e-30)
    recip = 1.0 / big_s
    e = jnp.exp(s - m)
    attn_o[...] = e * recip
    t_new = t_s[...][:, :1] + jnp.sum(e * s, axis=-1, keepdims=True)
    acc_new = acc_s[...] + lax.dot_general(
        e, kn, (((1,), (0,)), ((), ())), preferred_element_type=jnp.float32)
    t_s[...] = jnp.broadcast_to(t_new, t_s.shape)
    acc_s[...] = acc_new

    @pl.when(i == nkb - 1)
    def _fin():
        msem_o[...] = acc_new * recip
        max_attn = recip
        entropy = m + jnp.log(big_s) - t_new * recip
        max_entropy = math.log(kk + _EPS)
        norm_ent = jnp.clip(entropy / (max_entropy + _EPS), 0.0, 1.0)
        gap = (1.0 - jnp.exp(r2_ref[...][:, :1] - m)) * recip
        conf = jnp.clip(_W1 * max_attn + _W2 * (1.0 - norm_ent) + _W3 * gap,
                        0.0, 1.0)
        conf_o[...] = jnp.broadcast_to(conf, conf_o.shape)


def _pass3_body(cv_ref, cand_ref, tail_ref, topk_o, *, kk):
    n = cv_ref.shape[0]
    w = cv_ref.shape[1]
    c = lax.broadcasted_iota(jnp.int32, (n, w), 1)
    cb = c // _CB
    cw = c - cb * _CB
    cand_sel = jnp.zeros((n, w), jnp.int32)
    for j in range(_TOP_K):
        cand_sel = jnp.where(
            cb == j, jnp.broadcast_to(cand_ref[...][:, j:j + 1], (n, w)),
            cand_sel)
    clamp = ((kk - _CB) // 128) * 128
    cand_sel = jnp.clip(cand_sel, 0, kk // _CB)
    gid = jnp.minimum(cand_sel * _CB, clamp) + cw
    vals = cv_ref[...]
    lane = lax.broadcasted_iota(jnp.int32, (n, 128), 1)
    tcol = (kk - 1) // 128
    tgid = tcol * 128 + lane
    tvals = jnp.where(tgid < kk, tail_ref[...], _NEG)
    vals = jnp.concatenate([vals, tvals], axis=1)
    gid = jnp.concatenate([gid, tgid], axis=1)
    out = jnp.zeros((n, 128), jnp.int32)
    for j in range(_TOP_K):
        _, ix = _top1(vals, gid)
        out = jnp.where(lane == j, jnp.broadcast_to(ix, (n, 128)), out)
        vals = jnp.where(gid == ix, _NEG, vals)
    topk_o[...] = out


def _sc_gather(cand, attn, n, kk):
    info = plsc.get_sparse_core_info()
    nw = info.num_cores * info.num_subcores
    rows_per_w = n // nw
    mesh = plsc.VectorSubcoreMesh(core_axis_name="c", subcore_axis_name="s")
    cand3 = cand.reshape(n, 1, 128)

    clamp = ((kk - _CB) // 128) * 128

    @functools.partial(
        pl.kernel, mesh=mesh,
        out_type=jax.ShapeDtypeStruct((n, _TOP_K, _CB), jnp.float32),
        scratch_types=[
            pltpu.VMEM((rows_per_w, 1, 128), jnp.int32),
            pltpu.VMEM((_TOP_K, 8, _CB), jnp.float32),
            pltpu.VMEM((_TOP_K, 8, _CB), jnp.float32),
            pltpu.VMEM((_TOP_K, _CB), jnp.float32),
            pltpu.SemaphoreType.DMA,
            pltpu.SemaphoreType.DMA,
        ],
    )
    def k(cand_hbm, attn_hbm, out_hbm, cand_v, vals_a, vals_b, row_v,
          sem_a, sem_b):
        wid = lax.axis_index("s") * info.num_cores + lax.axis_index("c")
        base = wid * rows_per_w
        pltpu.sync_copy(cand_hbm.at[pl.ds(base, rows_per_w)], cand_v)

        def start_row(t, buf, sem):
            r = base + t
            g8 = pl.multiple_of((r // 8) * 8, 8)
            cvec = jnp.maximum(cand_v[t, 0, pl.ds(0, 16)], 0)
            for j in range(_TOP_K):
                start = pl.multiple_of(
                    jnp.minimum(jnp.minimum(cvec[j], kk // _CB) * _CB, clamp),
                    128)
                pltpu.make_async_copy(
                    attn_hbm.at[pl.ds(g8, 8), pl.ds(start, _CB)],
                    buf.at[j], sem).start()

        def wait_row(buf, sem):
            for j in range(_TOP_K):
                pltpu.make_async_copy(
                    attn_hbm.at[pl.ds(0, 8), pl.ds(0, _CB)],
                    buf.at[j], sem).wait()

        def finish_row(t, buf):
            r = base + t
            rm8 = r - (r // 8) * 8

            def extract(i, _):
                for j in range(_TOP_K):
                    row_v[j, pl.ds(i * 16, 16)] = buf[j, rm8,
                                                      pl.ds(i * 16, 16)]
                return _

            lax.fori_loop(0, _CB // 16, extract, 0)
            pltpu.sync_copy(row_v, out_hbm.at[r])

        start_row(0, vals_a, sem_a)

        def body(h, carry):
            t0 = 2 * h
            start_row(t0 + 1, vals_b, sem_b)
            wait_row(vals_a, sem_a)
            finish_row(t0, vals_a)

            @pl.when(t0 + 2 < rows_per_w)
            def _refill():
                start_row(t0 + 2, vals_a, sem_a)

            wait_row(vals_b, sem_b)
            finish_row(t0 + 1, vals_b)
            return carry

        lax.fori_loop(0, rows_per_w // 2, body, 0)

    return k(cand3, attn)


def kernel(query, keys):
    n, d = query.shape
    kk = keys.shape[0]
    nkb = pl.cdiv(kk, _KB)
    kb1 = _KB1
    nkb1 = pl.cdiv(kk, kb1)

    p1 = pl.pallas_call(
        functools.partial(_pass1_body, kb=kb1, nkb=nkb1, kk=kk),
        grid=(nkb1,),
        in_specs=[
            pl.BlockSpec((n, d), lambda i: (0, 0)),
            pl.BlockSpec((kb1, d), lambda i: (i, 0)),
        ],
        out_specs=[
            pl.BlockSpec((n, 128), lambda i: (0, 0)),
            pl.BlockSpec((n, 128), lambda i: (0, 0)),
            pl.BlockSpec((n, 128), lambda i: (0, 0)),
            pl.BlockSpec((n, 128), lambda i: (0, 0)),
        ],
        out_shape=[
            jax.ShapeDtypeStruct((n, 128), jnp.float32),
            jax.ShapeDtypeStruct((n, 128), jnp.float32),
            jax.ShapeDtypeStruct((n, 128), jnp.float32),
            jax.ShapeDtypeStruct((n, 128), jnp.int32),
        ],
        scratch_shapes=[
            pltpu.VMEM((n, 128), jnp.float32),
            pltpu.VMEM((n, 128), jnp.float32),
            pltpu.VMEM((n, 128), jnp.float32),
            pltpu.VMEM((n, 128), jnp.float32),
        ],
    )
    m_row, s_row, r2_row, cand_r = p1(query, keys)

    attn, m_sem, conf_r = pl.pallas_call(
        functools.partial(_pass2_body, kb=_KB, nkb=nkb, kk=kk),
        grid=(nkb,),
        in_specs=[
            pl.BlockSpec((n, d), lambda i: (0, 0)),
            pl.BlockSpec((_KB, d), lambda i: (i, 0)),
            pl.BlockSpec((n, 128), lambda i: (0, 0)),
            pl.BlockSpec((n, 128), lambda i: (0, 0)),
            pl.BlockSpec((n, 128), lambda i: (0, 0)),
        ],
        out_specs=[
            pl.BlockSpec((n, _KB), lambda i: (0, i)),
            pl.BlockSpec((n, d), lambda i: (0, 0)),
            pl.BlockSpec((n, 128), lambda i: (0, 0)),
        ],
        out_shape=[
            jax.ShapeDtypeStruct((n, kk), jnp.float32),
            jax.ShapeDtypeStruct((n, d), jnp.float32),
            jax.ShapeDtypeStruct((n, 128), jnp.float32),
        ],
        scratch_shapes=[
            pltpu.VMEM((n, 128), jnp.float32),
            pltpu.VMEM((n, d), jnp.float32),
        ],
    )(query, keys, m_row, s_row, r2_row)

    cand_vals = _sc_gather(cand_r, attn, n, kk).reshape(n, _TOP_K * _CB)

    rb = 128
    tcol = (kk - 1) // 128
    topk_r = pl.pallas_call(
        functools.partial(_pass3_body, kk=kk),
        grid=(n // rb,),
        in_specs=[
            pl.BlockSpec((rb, _TOP_K * _CB), lambda i: (i, 0)),
            pl.BlockSpec((rb, 128), lambda i: (i, 0)),
            pl.BlockSpec((rb, 128), lambda i: (i, tcol)),
        ],
        out_specs=pl.BlockSpec((rb, 128), lambda i: (i, 0)),
        out_shape=jax.ShapeDtypeStruct((n, 128), jnp.int32),
    )(cand_vals, cand_r, attn)

    conf = conf_r[:, 0]
    topk_idx = topk_r[:, :_TOP_K]
    return (m_sem, attn, conf, topk_idx)


# speedup vs baseline: 1.0180x; 1.0008x over previous
import functools
import math

import jax
import jax.numpy as jnp
from jax import lax
from jax.experimental import pallas as pl
from jax.experimental.pallas import tpu as pltpu
from jax.experimental.pallas import tpu_sc as plsc

_TOP_K = 5
_W1, _W2, _W3 = 0.5, 0.3, 0.2
_EPS = 1e-12
_NEG = -1e30
_IMAX = 2**31 - 1

_KB = 2048
_KB1 = 3072
_CB = 1024


def _normalize(x):
    n = jnp.sqrt(jnp.sum(x * x, axis=-1, keepdims=True))
    return x / jnp.maximum(n, 1e-12)


def _top1(vals, ids):
    v = jnp.max(vals, axis=-1, keepdims=True)
    cand = jnp.where(vals >= v, ids, _IMAX)
    ix = jnp.min(cand, axis=-1, keepdims=True)
    return v, ix


def _pass1_body(q_ref, k_ref, m_o, s_o, r2_o, cand_o,
                m_s, l_s, r2_s, bm_s, *, kb, nkb, kk):
    i = pl.program_id(0)
    n = q_ref.shape[0]

    @pl.when(i == 0)
    def _init():
        m_s[...] = jnp.full_like(m_s[...], _NEG)
        l_s[...] = jnp.zeros_like(l_s[...])
        r2_s[...] = jnp.full_like(r2_s[...], _NEG)
        bm_s[...] = jnp.full_like(bm_s[...], _NEG)

    qn = _normalize(q_ref[...])
    k = k_ref[...]
    rows_left = kk - i * kb
    rmask = lax.broadcasted_iota(jnp.int32, (kb, 1), 0) < rows_left
    k = jnp.where(rmask, k, 0.0)
    kn = _normalize(k)
    s = lax.dot_general(qn, kn, (((1,), (1,)), ((), ())),
                        preferred_element_type=jnp.float32)
    col = lax.broadcasted_iota(jnp.int32, (n, kb), 1) + i * kb
    s = jnp.where(col < kk, s, _NEG)

    nsub = kb // _CB
    bms = [jnp.max(s[:, j * _CB:(j + 1) * _CB], axis=-1, keepdims=True)
           for j in range(nsub)]
    lane = lax.broadcasted_iota(jnp.int32, (n, 128), 1)
    bm = bm_s[...]
    for j in range(nsub):
        bm = jnp.where(lane == nsub * i + j,
                       jnp.broadcast_to(bms[j], bm.shape), bm)
    bm_s[...] = bm

    m_old = m_s[...][:, :1]
    bmax = bms[0]
    for j in range(1, nsub):
        bmax = jnp.maximum(bmax, bms[j])
    m_new = jnp.maximum(m_old, bmax)
    alpha = jnp.exp(m_old - m_new)
    p = jnp.exp(s - m_new)
    l_new = l_s[...][:, :1] * alpha + jnp.sum(p, axis=-1, keepdims=True)

    m_s[...] = jnp.broadcast_to(m_new, m_s.shape)
    l_s[...] = jnp.broadcast_to(l_new, l_s.shape)

    eq = s >= bmax
    v2m = jnp.max(jnp.where(eq, _NEG, s), axis=-1, keepdims=True)
    cnt = jnp.sum(jnp.where(eq, 1.0, 0.0), axis=-1, keepdims=True)
    v2b = jnp.where(cnt > 1.0, bmax, v2m)
    r2_old = r2_s[...][:, :1]
    r2_new = jnp.maximum(jnp.minimum(m_old, bmax), jnp.maximum(r2_old, v2b))
    r2_s[...] = jnp.broadcast_to(r2_new, r2_s.shape)

    @pl.when(i == nkb - 1)
    def _fin():
        m_o[...] = jnp.broadcast_to(m_new, m_o.shape)
        s_o[...] = jnp.broadcast_to(l_new, s_o.shape)
        r2_o[...] = jnp.broadcast_to(r2_new, r2_o.shape)

        bvals = bm_s[...]
        bids = lane
        sel = []
        for _ in range(_TOP_K):
            _, ix = _top1(bvals, bids)
            sel.append(ix)
            bvals = jnp.where(bids == ix, _NEG, bvals)
        cand = jnp.zeros((n, 128), jnp.int32)
        for j in range(_TOP_K):
            cand = jnp.where(lane == j, jnp.broadcast_to(sel[j], (n, 128)),
                             cand)
        cand_o[...] = cand


def _pass2_body(q_ref, k_ref, m_ref, s_ref, r2_ref, attn_o, msem_o, conf_o,
                t_s, acc_s, *, kb, nkb, kk):
    i = pl.program_id(0)
    n = q_ref.shape[0]

    @pl.when(i == 0)
    def _init():
        t_s[...] = jnp.zeros_like(t_s[...])
        acc_s[...] = jnp.zeros_like(acc_s[...])

    qn = _normalize(q_ref[...])
    k = k_ref[...]
    rows_left = kk - i * kb
    rmask = lax.broadcasted_iota(jnp.int32, (kb, 1), 0) < rows_left
    k = jnp.where(rmask, k, 0.0)
    kn = _normalize(k)
    s = lax.dot_general(qn, kn, (((1,), (1,)), ((), ())),
                        preferred_element_type=jnp.float32)
    col = lax.broadcasted_iota(jnp.int32, (n, kb), 1) + i * kb
    s = jnp.where(col < kk, s, _NEG)
    m = m_ref[...][:, :1]
    big_s = jnp.maximum(s_ref[...][:, :1], 1e-30)
    recip = 1.0 / big_s
    e = jnp.exp(s - m)
    attn_o[...] = e * recip
    t_new = t_s[...][:, :1] + jnp.sum(e * s, axis=-1, keepdims=True)
    acc_new = acc_s[...] + lax.dot_general(
        e, kn, (((1,), (0,)), ((), ())), preferred_element_type=jnp.float32)
    t_s[...] = jnp.broadcast_to(t_new, t_s.shape)
    acc_s[...] = acc_new

    @pl.when(i == nkb - 1)
    def _fin():
        msem_o[...] = acc_new * recip
        max_attn = recip
        entropy = m + jnp.log(big_s) - t_new * recip
        max_entropy = math.log(kk + _EPS)
        norm_ent = jnp.clip(entropy / (max_entropy + _EPS), 0.0, 1.0)
        gap = (1.0 - jnp.exp(r2_ref[...][:, :1] - m)) * recip
        conf = jnp.clip(_W1 * max_attn + _W2 * (1.0 - norm_ent) + _W3 * gap,
                        0.0, 1.0)
        conf_o[...] = jnp.broadcast_to(conf, conf_o.shape)


def _pass3_body(cv_ref, cand_ref, tail_ref, topk_o, *, kk):
    n = cv_ref.shape[0]
    w = cv_ref.shape[1]
    c = lax.broadcasted_iota(jnp.int32, (n, w), 1)
    cb = c // _CB
    cw = c - cb * _CB
    cand_sel = jnp.zeros((n, w), jnp.int32)
    for j in range(_TOP_K):
        cand_sel = jnp.where(
            cb == j, jnp.broadcast_to(cand_ref[...][:, j:j + 1], (n, w)),
            cand_sel)
    clamp = ((kk - _CB) // 128) * 128
    cand_sel = jnp.clip(cand_sel, 0, kk // _CB)
    gid = jnp.minimum(cand_sel * _CB, clamp) + cw
    vals = cv_ref[...]
    lane = lax.broadcasted_iota(jnp.int32, (n, 128), 1)
    tcol = (kk - 1) // 128
    tgid = tcol * 128 + lane
    tvals = jnp.where(tgid < kk, tail_ref[...], _NEG)
    vals = jnp.concatenate([vals, tvals], axis=1)
    gid = jnp.concatenate([gid, tgid], axis=1)
    out = jnp.zeros((n, 128), jnp.int32)
    for j in range(_TOP_K):
        _, ix = _top1(vals, gid)
        out = jnp.where(lane == j, jnp.broadcast_to(ix, (n, 128)), out)
        vals = jnp.where(gid == ix, _NEG, vals)
    topk_o[...] = out


def _sc_gather(cand, attn, n, kk):
    info = plsc.get_sparse_core_info()
    nw = info.num_cores * info.num_subcores
    rows_per_w = n // nw
    mesh = plsc.VectorSubcoreMesh(core_axis_name="c", subcore_axis_name="s")
    cand3 = cand.reshape(n, 1, 128)

    clamp = ((kk - _CB) // 128) * 128

    @functools.partial(
        pl.kernel, mesh=mesh,
        out_type=jax.ShapeDtypeStruct((n, _TOP_K, _CB), jnp.float32),
        scratch_types=[
            pltpu.VMEM((rows_per_w, 1, 128), jnp.int32),
            pltpu.VMEM((_TOP_K, 8, _CB), jnp.float32),
            pltpu.VMEM((_TOP_K, 8, _CB), jnp.float32),
            pltpu.VMEM((_TOP_K, _CB), jnp.float32),
            pltpu.SemaphoreType.DMA,
            pltpu.SemaphoreType.DMA,
        ],
    )
    def k(cand_hbm, attn_hbm, out_hbm, cand_v, vals_a, vals_b, row_v,
          sem_a, sem_b):
        wid = lax.axis_index("s") * info.num_cores + lax.axis_index("c")
        base = wid * rows_per_w
        pltpu.sync_copy(cand_hbm.at[pl.ds(base, rows_per_w)], cand_v)

        def start_row(t, buf, sem):
            r = base + t
            g8 = pl.multiple_of((r // 8) * 8, 8)
            cvec = jnp.maximum(cand_v[t, 0, pl.ds(0, 16)], 0)
            for j in range(_TOP_K):
                start = pl.multiple_of(
                    jnp.minimum(jnp.minimum(cvec[j], kk // _CB) * _CB, clamp),
                    128)
                pltpu.make_async_copy(
                    attn_hbm.at[pl.ds(g8, 8), pl.ds(start, _CB)],
                    buf.at[j], sem).start()

        def wait_row(buf, sem):
            for j in range(_TOP_K):
                pltpu.make_async_copy(
                    attn_hbm.at[pl.ds(0, 8), pl.ds(0, _CB)],
                    buf.at[j], sem).wait()

        def finish_row(t, buf):
            r = base + t
            rm8 = r - (r // 8) * 8

            def extract(i, _):
                for j in range(_TOP_K):
                    row_v[j, pl.ds(i * 16, 16)] = buf[j, rm8,
                                                      pl.ds(i * 16, 16)]
                return _

            lax.fori_loop(0, _CB // 16, extract, 0)
            pltpu.sync_copy(row_v, out_hbm.at[r])

        start_row(0, vals_a, sem_a)

        def body(h, carry):
            t0 = 2 * h
            start_row(t0 + 1, vals_b, sem_b)
            wait_row(vals_a, sem_a)
            finish_row(t0, vals_a)

            @pl.when(t0 + 2 < rows_per_w)
            def _refill():
                start_row(t0 + 2, vals_a, sem_a)

            wait_row(vals_b, sem_b)
            finish_row(t0 + 1, vals_b)
            return carry

        lax.fori_loop(0, rows_per_w // 2, body, 0)

    return k(cand3, attn)


def kernel(query, keys):
    n, d = query.shape
    kk = keys.shape[0]
    nkb = pl.cdiv(kk, _KB)
    kb1 = _KB1
    nkb1 = pl.cdiv(kk, kb1)

    p1 = pl.pallas_call(
        functools.partial(_pass1_body, kb=kb1, nkb=nkb1, kk=kk),
        grid=(nkb1,),
        in_specs=[
            pl.BlockSpec((n, d), lambda i: (0, 0)),
            pl.BlockSpec((kb1, d), lambda i: (i, 0)),
        ],
        out_specs=[
            pl.BlockSpec((n, 128), lambda i: (0, 0)),
            pl.BlockSpec((n, 128), lambda i: (0, 0)),
            pl.BlockSpec((n, 128), lambda i: (0, 0)),
            pl.BlockSpec((n, 128), lambda i: (0, 0)),
        ],
        out_shape=[
            jax.ShapeDtypeStruct((n, 128), jnp.float32),
            jax.ShapeDtypeStruct((n, 128), jnp.float32),
            jax.ShapeDtypeStruct((n, 128), jnp.float32),
            jax.ShapeDtypeStruct((n, 128), jnp.int32),
        ],
        scratch_shapes=[
            pltpu.VMEM((n, 128), jnp.float32),
            pltpu.VMEM((n, 128), jnp.float32),
            pltpu.VMEM((n, 128), jnp.float32),
            pltpu.VMEM((n, 128), jnp.float32),
        ],
    )
    m_row, s_row, r2_row, cand_r = p1(query, keys)

    attn, m_sem, conf_r = pl.pallas_call(
        functools.partial(_pass2_body, kb=_KB, nkb=nkb, kk=kk),
        grid=(nkb,),
        in_specs=[
            pl.BlockSpec((n, d), lambda i: (0, 0)),
            pl.BlockSpec((_KB, d), lambda i: (i, 0)),
            pl.BlockSpec((n, 128), lambda i: (0, 0)),
            pl.BlockSpec((n, 128), lambda i: (0, 0)),
            pl.BlockSpec((n, 128), lambda i: (0, 0)),
        ],
        out_specs=[
            pl.BlockSpec((n, _KB), lambda i: (0, i)),
            pl.BlockSpec((n, d), lambda i: (0, 0)),
            pl.BlockSpec((n, 128), lambda i: (0, 0)),
        ],
        out_shape=[
            jax.ShapeDtypeStruct((n, kk), jnp.float32),
            jax.ShapeDtypeStruct((n, d), jnp.float32),
            jax.ShapeDtypeStruct((n, 128), jnp.float32),
        ],
        scratch_shapes=[
            pltpu.VMEM((n, 128), jnp.float32),
            pltpu.VMEM((n, d), jnp.float32),
        ],
    )(query, keys, m_row, s_row, r2_row)

    cand_vals = _sc_gather(cand_r, attn, n, kk).reshape(n, _TOP_K * _CB)

    rb = 128
    tcol = (kk - 1) // 128
    topk_r = pl.pallas_call(
        functools.partial(_pass3_body, kk=kk),
        grid=(n // rb,),
        in_specs=[
            pl.BlockSpec((rb, _TOP_K * _CB), lambda i: (i, 0)),
            pl.BlockSpec((rb, 128), lambda i: (i, 0)),
            pl.BlockSpec((rb, 128), lambda i: (i, tcol)),
        ],
        out_specs=pl.BlockSpec((rb, 128), lambda i: (i, 0)),
        out_shape=jax.ShapeDtypeStruct((n, 128), jnp.int32),
    )(cand_vals, cand_r, attn)

    conf = conf_r[:, 0]
    topk_idx = topk_r[:, :_TOP_K]
    return (m_sem, attn, conf, topk_idx)
